# baseline (speedup 1.0000x reference)
import functools

import jax
import jax.numpy as jnp
from jax import lax
from jax.experimental import pallas as pl
from jax.experimental.pallas import tpu as pltpu
from jax.experimental.pallas import tpu_sc as plsc

VOCAB = 100000
EMBED = 128
BATCH = 16384

NC = 2
NS = 16
NW = NC * NS
B_PER_W = BATCH // NW
CHUNK = 128
NCHUNK = B_PER_W // CHUNK
GROUPS = CHUNK // 16


def _sc_inner_kernel(a_hbm, b_hbm, emb_hbm, ctx_hbm, out_hbm,
                     idx_a, idx_b, rows_a, rows_b, inner_v, sem_a, sem_b):
    wid = lax.axis_index("s") * NC + lax.axis_index("c")
    base = wid * B_PER_W

    def chunk_body(j, carry):
        cb = base + j * CHUNK
        pltpu.sync_copy(a_hbm.at[pl.ds(cb, CHUNK)], idx_a)
        pltpu.sync_copy(b_hbm.at[pl.ds(cb, CHUNK)], idx_b)
        cp_a = pltpu.async_copy(emb_hbm.at[idx_a], rows_a, sem_a)
        cp_b = pltpu.async_copy(ctx_hbm.at[idx_b], rows_b, sem_b)
        cp_a.wait()
        cp_b.wait()

        def group_body(g, carry2):
            rid = g * 16 + lax.iota(jnp.int32, 16)
            acc = jnp.zeros((16,), jnp.float32)
            for k in range(EMBED):
                kk = jnp.full((16,), k, jnp.int32)
                va = plsc.load_gather(rows_a, [rid, kk])
                vb = plsc.load_gather(rows_b, [rid, kk])
                acc = acc + va * vb
            inner_v[pl.ds(g * 16, 16)] = acc
            return carry2

        lax.fori_loop(0, GROUPS, group_body, 0, unroll=False)
        pltpu.sync_copy(inner_v, out_hbm.at[pl.ds(cb, CHUNK)])
        return carry

    lax.fori_loop(0, NCHUNK, chunk_body, 0, unroll=False)


@jax.jit
def _sc_inner(a, b, embeddings, context_embeddings):
    mesh = plsc.VectorSubcoreMesh(core_axis_name="c", subcore_axis_name="s")
    kern = pl.kernel(
        _sc_inner_kernel,
        out_type=jax.ShapeDtypeStruct((BATCH,), jnp.float32),
        mesh=mesh,
        compiler_params=pltpu.CompilerParams(needs_layout_passes=False),
        scratch_types=[
            pltpu.VMEM((CHUNK,), jnp.int32),
            pltpu.VMEM((CHUNK,), jnp.int32),
            pltpu.VMEM((CHUNK, EMBED), jnp.float32),
            pltpu.VMEM((CHUNK, EMBED), jnp.float32),
            pltpu.VMEM((CHUNK,), jnp.float32),
            pltpu.SemaphoreType.DMA,
            pltpu.SemaphoreType.DMA,
        ],
    )
    return kern(a, b, embeddings, context_embeddings)


def _loss_body(inner_ref, sign_ref, out_ref):
    z = sign_ref[...] * inner_ref[...]
    ls = jnp.minimum(z, 0.0) - jnp.log1p(jnp.exp(-jnp.abs(z)))
    out_ref[0, 0] = -jnp.sum(ls) / BATCH


@jax.jit
def _loss(inner, sign):
    res = pl.pallas_call(
        _loss_body,
        out_shape=jax.ShapeDtypeStruct((1, 1), jnp.float32),
        in_specs=[
            pl.BlockSpec(memory_space=pltpu.VMEM),
            pl.BlockSpec(memory_space=pltpu.VMEM),
        ],
        out_specs=pl.BlockSpec(memory_space=pltpu.SMEM),
    )(inner.reshape(128, 128), sign.reshape(128, 128))
    return res[0, 0]


def kernel(a, b, sign, embeddings, context_embeddings):
    inner = _sc_inner(a, b, embeddings, context_embeddings)
    return _loss(inner, sign)


# speedup vs baseline: 2.1694x; 2.1694x over previous
import functools

import jax
import jax.numpy as jnp
from jax import lax
from jax.experimental import pallas as pl
from jax.experimental.pallas import tpu as pltpu
from jax.experimental.pallas import tpu_sc as plsc

VOCAB = 100000
EMBED = 128
BATCH = 16384

NC = 2
NS = 16
NW = NC * NS
B_PER_W = BATCH // NW
CHUNK = 128
NCHUNK = B_PER_W // CHUNK
GROUPS = CHUNK // 16


def _sc_inner_kernel(a_hbm, b_hbm, emb_hbm, ctx_hbm, out_hbm,
                     idx_a, idx_b, rows_a, rows_b, stage, inner_v, sem_a, sem_b):
    wid = lax.axis_index("s") * NC + lax.axis_index("c")
    base = wid * B_PER_W
    iota16 = lax.iota(jnp.int32, 16)

    def chunk_body(j, carry):
        cb = base + j * CHUNK
        pltpu.sync_copy(a_hbm.at[pl.ds(cb, CHUNK)], idx_a)
        pltpu.sync_copy(b_hbm.at[pl.ds(cb, CHUNK)], idx_b)
        cp_a = pltpu.async_copy(emb_hbm.at[idx_a], rows_a, sem_a)
        cp_b = pltpu.async_copy(ctx_hbm.at[idx_b], rows_b, sem_b)
        cp_a.wait()
        cp_b.wait()

        def group_body(g, carry2):
            for rr in range(16):
                r = g * 16 + rr
                acc = None
                for c in range(EMBED // 16):
                    va = rows_a[r, pl.ds(c * 16, 16)]
                    vb = rows_b[r, pl.ds(c * 16, 16)]
                    p = va * vb
                    acc = p if acc is None else acc + p
                stage[rr, :] = acc
            s = None
            for l in range(16):
                col = plsc.load_gather(stage, [iota16, jnp.full((16,), l, jnp.int32)])
                s = col if s is None else s + col
            inner_v[pl.ds(g * 16, 16)] = s
            return carry2

        lax.fori_loop(0, GROUPS, group_body, 0, unroll=False)
        pltpu.sync_copy(inner_v, out_hbm.at[pl.ds(cb, CHUNK)])
        return carry

    lax.fori_loop(0, NCHUNK, chunk_body, 0, unroll=False)


@jax.jit
def _sc_inner(a, b, embeddings, context_embeddings):
    mesh = plsc.VectorSubcoreMesh(core_axis_name="c", subcore_axis_name="s")
    kern = pl.kernel(
        _sc_inner_kernel,
        out_type=jax.ShapeDtypeStruct((BATCH,), jnp.float32),
        mesh=mesh,
        compiler_params=pltpu.CompilerParams(needs_layout_passes=False),
        scratch_types=[
            pltpu.VMEM((CHUNK,), jnp.int32),
            pltpu.VMEM((CHUNK,), jnp.int32),
            pltpu.VMEM((CHUNK, EMBED), jnp.float32),
            pltpu.VMEM((CHUNK, EMBED), jnp.float32),
            pltpu.VMEM((16, 16), jnp.float32),
            pltpu.VMEM((CHUNK,), jnp.float32),
            pltpu.SemaphoreType.DMA,
            pltpu.SemaphoreType.DMA,
        ],
    )
    return kern(a, b, embeddings, context_embeddings)


def _loss_body(inner_ref, sign_ref, out_ref):
    z = sign_ref[...] * inner_ref[...]
    ls = jnp.minimum(z, 0.0) - jnp.log1p(jnp.exp(-jnp.abs(z)))
    out_ref[0, 0] = -jnp.sum(ls) / BATCH


@jax.jit
def _loss(inner, sign):
    res = pl.pallas_call(
        _loss_body,
        out_shape=jax.ShapeDtypeStruct((1, 1), jnp.float32),
        in_specs=[
            pl.BlockSpec(memory_space=pltpu.VMEM),
            pl.BlockSpec(memory_space=pltpu.VMEM),
        ],
        out_specs=pl.BlockSpec(memory_space=pltpu.SMEM),
    )(inner.reshape(128, 128), sign.reshape(128, 128))
    return res[0, 0]


def kernel(a, b, sign, embeddings, context_embeddings):
    inner = _sc_inner(a, b, embeddings, context_embeddings)
    return _loss(inner, sign)


# speedup vs baseline: 2.4504x; 1.1296x over previous
import functools

import jax
import jax.numpy as jnp
from jax import lax
from jax.experimental import pallas as pl
from jax.experimental.pallas import tpu as pltpu
from jax.experimental.pallas import tpu_sc as plsc

VOCAB = 100000
EMBED = 128
BATCH = 16384

NC = 2
NS = 16
NW = NC * NS
B_PER_W = BATCH // NW
CHUNK = 128
NCHUNK = B_PER_W // CHUNK
GROUPS = CHUNK // 16


def _sc_inner_kernel(a_hbm, b_hbm, emb_hbm, ctx_hbm, out_hbm,
                     idx_a, idx_b, rows_a, rows_b, stage, inner_v,
                     sem_a0, sem_a1, sem_b0, sem_b1, sem_out):
    wid = lax.axis_index("s") * NC + lax.axis_index("c")
    base = wid * B_PER_W
    iota16 = lax.iota(jnp.int32, 16)
    sems_a = [sem_a0, sem_a1]
    sems_b = [sem_b0, sem_b1]

    def start_chunk(j, buf):
        cb = base + j * CHUNK
        pltpu.sync_copy(a_hbm.at[pl.ds(cb, CHUNK)], idx_a.at[buf])
        pltpu.sync_copy(b_hbm.at[pl.ds(cb, CHUNK)], idx_b.at[buf])
        cp_a = pltpu.async_copy(emb_hbm.at[idx_a.at[buf]], rows_a.at[buf], sems_a[buf])
        cp_b = pltpu.async_copy(ctx_hbm.at[idx_b.at[buf]], rows_b.at[buf], sems_b[buf])
        return cp_a, cp_b

    def compute_chunk(j, buf):
        ra = rows_a.at[buf]
        rb = rows_b.at[buf]

        def group_body(g, carry2):
            for rr in range(16):
                r = g * 16 + rr
                acc = None
                for c in range(EMBED // 16):
                    va = ra[r, pl.ds(c * 16, 16)]
                    vb = rb[r, pl.ds(c * 16, 16)]
                    p = va * vb
                    acc = p if acc is None else acc + p
                stage[rr, :] = acc
            s = None
            for l in range(16):
                col = plsc.load_gather(stage, [iota16, jnp.full((16,), l, jnp.int32)])
                s = col if s is None else s + col
            inner_v[j, pl.ds(g * 16, 16)] = s
            return carry2

        lax.fori_loop(0, GROUPS, group_body, 0, unroll=False)

    cps = start_chunk(0, 0)
    out_cps = []
    for j in range(NCHUNK):
        buf = j % 2
        nxt = None
        if j + 1 < NCHUNK:
            nxt = start_chunk(j + 1, (j + 1) % 2)
        cps[0].wait()
        cps[1].wait()
        compute_chunk(j, buf)
        cb = base + j * CHUNK
        out_cps.append(
            pltpu.async_copy(inner_v.at[j], out_hbm.at[pl.ds(cb, CHUNK)], sem_out))
        cps = nxt
    for cp in out_cps:
        cp.wait()


@jax.jit
def _sc_inner(a, b, embeddings, context_embeddings):
    mesh = plsc.VectorSubcoreMesh(core_axis_name="c", subcore_axis_name="s")
    kern = pl.kernel(
        _sc_inner_kernel,
        out_type=jax.ShapeDtypeStruct((BATCH,), jnp.float32),
        mesh=mesh,
        compiler_params=pltpu.CompilerParams(needs_layout_passes=False),
        scratch_types=[
            pltpu.VMEM((2, CHUNK), jnp.int32),
            pltpu.VMEM((2, CHUNK), jnp.int32),
            pltpu.VMEM((2, CHUNK, EMBED), jnp.float32),
            pltpu.VMEM((2, CHUNK, EMBED), jnp.float32),
            pltpu.VMEM((16, 16), jnp.float32),
            pltpu.VMEM((NCHUNK, CHUNK), jnp.float32),
            pltpu.SemaphoreType.DMA,
            pltpu.SemaphoreType.DMA,
            pltpu.SemaphoreType.DMA,
            pltpu.SemaphoreType.DMA,
            pltpu.SemaphoreType.DMA,
        ],
    )
    return kern(a, b, embeddings, context_embeddings)


def _loss_body(inner_ref, sign_ref, out_ref):
    z = sign_ref[...] * inner_ref[...]
    ls = jnp.minimum(z, 0.0) - jnp.log1p(jnp.exp(-jnp.abs(z)))
    out_ref[0, 0] = -jnp.sum(ls) / BATCH


@jax.jit
def _loss(inner, sign):
    res = pl.pallas_call(
        _loss_body,
        out_shape=jax.ShapeDtypeStruct((1, 1), jnp.float32),
        in_specs=[
            pl.BlockSpec(memory_space=pltpu.VMEM),
            pl.BlockSpec(memory_space=pltpu.VMEM),
        ],
        out_specs=pl.BlockSpec(memory_space=pltpu.SMEM),
    )(inner.reshape(128, 128), sign.reshape(128, 128))
    return res[0, 0]


def kernel(a, b, sign, embeddings, context_embeddings):
    inner = _sc_inner(a, b, embeddings, context_embeddings)
    return _loss(inner, sign)
